# baseline (speedup 1.0000x reference)
import jax
import jax.numpy as jnp
from jax import lax
from jax.experimental import pallas as pl
from jax.experimental.pallas import tpu as pltpu
from jax.experimental.pallas import tpu_sc as plsc

B = 16
L = 2048
D = 300
NC = 2
NS = 16
LANES = 16
GROUP = 1200
VPG = GROUP // LANES
CH_GROUPS = 32
CHF = CH_GROUPS * GROUP
FTOT = B * L * D
DP = 304
TAILF = 912


def _body(seq_hbm, len_hbm, out_hbm, buf, tailbuf, acc, partial, tmp, obuf,
          len_vm, shared):
    c = lax.axis_index("c")
    s = lax.axis_index("s")
    row = c * 8 + s // 2
    half = s % 2

    pltpu.sync_copy(len_hbm, len_vm.at[pl.ds(0, B)])
    n = len_vm[pl.ds(row, LANES)][0]

    half4 = ((n + 7) // 8) * 4
    p0 = half * half4
    p1 = jnp.minimum(n, p0 + half4)
    cnt = jnp.maximum(p1 - p0, 0)

    a0 = (row * L + p0) * D
    fg = cnt // 4
    nch = (fg + CH_GROUPS - 1) // CH_GROUPS

    zeros = jnp.zeros((LANES,), jnp.float32)
    for v in range(VPG + 1):
        acc[pl.ds(v * LANES, LANES)] = zeros

    def chunk_body(ci, _):
        g_off = ci * CH_GROUPS
        g_cnt = jnp.minimum(fg - g_off, CH_GROUPS)
        s_f = a0 + g_off * GROUP
        d_f = jnp.minimum(s_f, FTOT - CHF)
        pltpu.sync_copy(seq_hbm.at[pl.ds(d_f, CHF)], buf)
        shift = s_f - d_f

        def group_body(g, _):
            base = shift + g * GROUP
            for v in range(VPG):
                x = buf[pl.ds(base + v * LANES, LANES)]
                plsc.addupdate(acc.at[pl.ds(v * LANES, LANES)], x)
            return 0

        lax.fori_loop(0, g_cnt, group_body, 0)
        return 0

    lax.fori_loop(0, nch, chunk_body, 0)

    t = (cnt - fg * 4) * D
    td_s = a0 + fg * GROUP
    td = jnp.minimum(td_s, FTOT - TAILF)
    pltpu.sync_copy(seq_hbm.at[pl.ds(td, TAILF)], tailbuf)
    tshift = td_s - td
    lane = lax.iota(jnp.int32, LANES)

    def tail_body(v, _):
        f = v * LANES
        m = (f + lane) < t
        x = jnp.where(m, tailbuf[pl.ds(tshift + f, LANES)], 0.0)
        plsc.addupdate(acc.at[pl.ds(f, LANES)], x)
        return 0

    lax.fori_loop(0, (t + LANES - 1) // LANES, tail_body, 0)

    for v in range(DP // LANES):
        r0 = acc[pl.ds(v * LANES, LANES)]
        r1 = acc[pl.ds(D + v * LANES, LANES)]
        r2 = acc[pl.ds(2 * D + v * LANES, LANES)]
        r3 = acc[pl.ds(3 * D + v * LANES, LANES)]
        partial[pl.ds(v * LANES, LANES)] = (r0 + r1) + (r2 + r3)

    pltpu.sync_copy(partial, shared.at[pl.ds(s * DP, DP)])
    plsc.subcore_barrier()

    @pl.when(half == 0)
    def _():
        pltpu.sync_copy(shared.at[pl.ds((s + 1) * DP, DP)], tmp)
        nv = jnp.full((LANES,), n.astype(jnp.float32), jnp.float32)
        for v in range(DP // LANES):
            a = partial[pl.ds(v * LANES, LANES)]
            b = tmp[pl.ds(v * LANES, LANES)]
            obuf[pl.ds(v * LANES, LANES)] = (a + b) / nv
        pltpu.sync_copy(obuf, out_hbm.at[row])


def _mean_sc(flat, len32):
    mesh = plsc.VectorSubcoreMesh(
        core_axis_name="c", subcore_axis_name="s", num_cores=NC,
        num_subcores=NS)
    return pl.kernel(
        _body,
        out_type=jax.ShapeDtypeStruct((B, DP), jnp.float32),
        mesh=mesh,
        scratch_types=[
            pltpu.VMEM((CHF,), jnp.float32),
            pltpu.VMEM((TAILF,), jnp.float32),
            pltpu.VMEM((GROUP + LANES,), jnp.float32),
            pltpu.VMEM((DP,), jnp.float32),
            pltpu.VMEM((DP,), jnp.float32),
            pltpu.VMEM((DP,), jnp.float32),
            pltpu.VMEM((2 * B,), jnp.int32),
            pltpu.VMEM_SHARED((NS * DP,), jnp.float32),
        ],
    )(flat, len32)


def kernel(sequences, lengths):
    flat = sequences.reshape(-1)
    len32 = lengths.astype(jnp.int32)
    out = _mean_sc(flat, len32)
    return out[:, :D]


# speedup vs baseline: 1.4103x; 1.4103x over previous
import jax
import jax.numpy as jnp
from jax import lax
from jax.experimental import pallas as pl
from jax.experimental.pallas import tpu as pltpu
from jax.experimental.pallas import tpu_sc as plsc

B = 16
L = 2048
D = 300
NC = 2
NS = 16
LANES = 16
GROUP = 1200
VPG = GROUP // LANES
NA = 38
CH_GROUPS = 32
CHF = CH_GROUPS * GROUP
FTOT = B * L * D
DP = 304
TAILF = 912


def _body(seq_hbm, len_hbm, out_hbm, buf0, buf1, tailbuf, acc, partial, tmp,
          obuf, len_vm, shared, sem0, sem1):
    c = lax.axis_index("c")
    s = lax.axis_index("s")
    row = c * 8 + s // 2
    half = s % 2

    pltpu.sync_copy(len_hbm, len_vm.at[pl.ds(0, B)])
    n = len_vm[pl.ds(row, LANES)][0]

    half4 = ((n + 7) // 8) * 4
    p0 = half * half4
    p1 = jnp.minimum(n, p0 + half4)
    cnt = jnp.maximum(p1 - p0, 0)

    a0 = (row * L + p0) * D
    fg = cnt // 4
    nch = (fg + CH_GROUPS - 1) // CH_GROUPS

    zeros = jnp.zeros((LANES,), jnp.float32)
    for v in range(VPG + 1):
        acc[pl.ds(v * LANES, LANES)] = zeros

    def dma_start(j, buf, sem):
        @pl.when(j < nch)
        def _():
            d_f = jnp.minimum(a0 + j * CHF, FTOT - CHF)
            pltpu.async_copy(seq_hbm.at[pl.ds(d_f, CHF)], buf, sem)

    def dma_wait(j, buf, sem):
        @pl.when(j < nch)
        def _():
            pltpu.make_async_copy(seq_hbm.at[pl.ds(0, CHF)], buf, sem).wait()

    def compute(j, buf):
        @pl.when(j < nch)
        def _():
            g_cnt = jnp.minimum(fg - j * CH_GROUPS, CH_GROUPS)
            s_f = a0 + j * CHF
            shift = s_f - jnp.minimum(s_f, FTOT - CHF)
            for lo, nv in ((0, NA), (NA, VPG - NA)):
                def g_body(g, accs):
                    base = shift + g * GROUP + lo * LANES
                    return tuple(
                        accs[v] + buf[pl.ds(base + v * LANES, LANES)]
                        for v in range(nv))
                accs = lax.fori_loop(0, g_cnt, g_body, (zeros,) * nv)
                for v in range(nv):
                    plsc.addupdate(acc.at[pl.ds((lo + v) * LANES, LANES)],
                                   accs[v])

    dma_start(0, buf0, sem0)
    npairs = (nch + 1) // 2

    def pair_body(i2, _):
        j0 = 2 * i2
        dma_start(j0 + 1, buf1, sem1)
        dma_wait(j0, buf0, sem0)
        compute(j0, buf0)
        dma_start(j0 + 2, buf0, sem0)
        dma_wait(j0 + 1, buf1, sem1)
        compute(j0 + 1, buf1)
        return 0

    lax.fori_loop(0, npairs, pair_body, 0)

    t = (cnt - fg * 4) * D
    td_s = a0 + fg * GROUP
    td = jnp.minimum(td_s, FTOT - TAILF)
    pltpu.sync_copy(seq_hbm.at[pl.ds(td, TAILF)], tailbuf)
    tshift = td_s - td
    lane = lax.iota(jnp.int32, LANES)

    def tail_body(v, _):
        f = v * LANES
        m = (f + lane) < t
        x = jnp.where(m, tailbuf[pl.ds(tshift + f, LANES)], 0.0)
        plsc.addupdate(acc.at[pl.ds(f, LANES)], x)
        return 0

    lax.fori_loop(0, (t + LANES - 1) // LANES, tail_body, 0)

    for v in range(DP // LANES):
        r0 = acc[pl.ds(v * LANES, LANES)]
        r1 = acc[pl.ds(D + v * LANES, LANES)]
        r2 = acc[pl.ds(2 * D + v * LANES, LANES)]
        r3 = acc[pl.ds(3 * D + v * LANES, LANES)]
        partial[pl.ds(v * LANES, LANES)] = (r0 + r1) + (r2 + r3)

    pltpu.sync_copy(partial, shared.at[pl.ds(s * DP, DP)])
    plsc.subcore_barrier()

    @pl.when(half == 0)
    def _():
        pltpu.sync_copy(shared.at[pl.ds((s + 1) * DP, DP)], tmp)
        nv16 = jnp.full((LANES,), n.astype(jnp.float32), jnp.float32)
        for v in range(DP // LANES):
            a = partial[pl.ds(v * LANES, LANES)]
            b = tmp[pl.ds(v * LANES, LANES)]
            obuf[pl.ds(v * LANES, LANES)] = (a + b) / nv16
        pltpu.sync_copy(obuf, out_hbm.at[row])


def _mean_sc(flat, len32):
    mesh = plsc.VectorSubcoreMesh(
        core_axis_name="c", subcore_axis_name="s", num_cores=NC,
        num_subcores=NS)
    return pl.kernel(
        _body,
        out_type=jax.ShapeDtypeStruct((B, DP), jnp.float32),
        mesh=mesh,
        scratch_types=[
            pltpu.VMEM((CHF,), jnp.float32),
            pltpu.VMEM((CHF,), jnp.float32),
            pltpu.VMEM((TAILF,), jnp.float32),
            pltpu.VMEM((GROUP + LANES,), jnp.float32),
            pltpu.VMEM((DP,), jnp.float32),
            pltpu.VMEM((DP,), jnp.float32),
            pltpu.VMEM((DP,), jnp.float32),
            pltpu.VMEM((2 * B,), jnp.int32),
            pltpu.VMEM_SHARED((NS * DP,), jnp.float32),
            pltpu.SemaphoreType.DMA,
            pltpu.SemaphoreType.DMA,
        ],
    )(flat, len32)


def kernel(sequences, lengths):
    flat = sequences.reshape(-1)
    len32 = lengths.astype(jnp.int32)
    out = _mean_sc(flat, len32)
    return out[:, :D]


# speedup vs baseline: 2.3705x; 1.6809x over previous
import jax
import jax.numpy as jnp
from jax import lax
from jax.experimental import pallas as pl
from jax.experimental.pallas import tpu as pltpu
from jax.experimental.pallas import tpu_sc as plsc

B = 16
L = 2048
D = 300
NC = 2
NS = 16
LANES = 16
NV = 19
LAST = D - LANES
CHR = 128
DP = NV * LANES


def _body(seq_hbm, len_hbm, out_hbm, buf0, buf1, partial, tmp, obuf,
          len_vm, shared, sem0, sem1):
    c = lax.axis_index("c")
    s = lax.axis_index("s")
    row = c * 8 + s // 2
    half = s % 2

    pltpu.sync_copy(len_hbm, len_vm.at[pl.ds(0, B)])
    n = len_vm[pl.ds(row, LANES)][0]

    half8 = ((n + 15) // 16) * 8
    p0 = half * half8
    p1 = jnp.minimum(n, p0 + half8)
    cnt = jnp.maximum(p1 - p0, 0)
    base_r = row * L + p0
    nch = (cnt + CHR - 1) // CHR

    lane = lax.iota(jnp.int32, LANES)
    lmask = lane >= ((NV - 1) * LANES - LAST)

    def chunk_start(j):
        d_r = jnp.minimum(base_r + j * CHR, B * L - CHR)
        return d_r, (base_r + j * CHR) - d_r

    def dma_start(j, buf, sem):
        @pl.when(j < nch)
        def _():
            d_r, _ = chunk_start(j)
            pltpu.async_copy(seq_hbm.at[pl.ds(d_r, CHR), :], buf, sem)

    def dma_wait(j, buf, sem):
        @pl.when(j < nch)
        def _():
            pltpu.make_async_copy(seq_hbm.at[pl.ds(0, CHR), :], buf, sem).wait()

    def compute(j, buf, accs):
        _, shift = chunk_start(j)
        rcnt = jnp.clip(cnt - j * CHR, 0, CHR)

        def r_body(r, a):
            new = [a[v] + buf[r, pl.ds(v * LANES, LANES)]
                   for v in range(NV - 1)]
            xl = buf[r, pl.ds(LAST, LANES)]
            new.append(a[NV - 1] + jnp.where(lmask, xl, 0.0))
            return tuple(new)

        return lax.fori_loop(shift, shift + rcnt, r_body, accs)

    zeros = jnp.zeros((LANES,), jnp.float32)
    accs0 = (zeros,) * NV
    dma_start(0, buf0, sem0)
    npairs = (nch + 1) // 2

    def pair_body(i2, accs):
        j0 = 2 * i2
        dma_start(j0 + 1, buf1, sem1)
        dma_wait(j0, buf0, sem0)
        accs = compute(j0, buf0, accs)
        dma_start(j0 + 2, buf0, sem0)
        dma_wait(j0 + 1, buf1, sem1)
        return compute(j0 + 1, buf1, accs)

    accs = lax.fori_loop(0, npairs, pair_body, accs0)

    for v in range(NV):
        partial[pl.ds(v * LANES, LANES)] = accs[v]

    pltpu.sync_copy(partial, shared.at[pl.ds(s * DP, DP)])
    plsc.subcore_barrier()

    @pl.when(half == 0)
    def _():
        pltpu.sync_copy(shared.at[pl.ds((s + 1) * DP, DP)], tmp)
        nv16 = jnp.full((LANES,), n.astype(jnp.float32), jnp.float32)
        for v in range(NV):
            a = partial[pl.ds(v * LANES, LANES)]
            b = tmp[pl.ds(v * LANES, LANES)]
            obuf[pl.ds(v * LANES, LANES)] = (a + b) / nv16
        pltpu.sync_copy(obuf, out_hbm.at[row])


def _mean_sc(seq2d, len32):
    mesh = plsc.VectorSubcoreMesh(
        core_axis_name="c", subcore_axis_name="s", num_cores=NC,
        num_subcores=NS)
    return pl.kernel(
        _body,
        out_type=jax.ShapeDtypeStruct((B, DP), jnp.float32),
        mesh=mesh,
        scratch_types=[
            pltpu.VMEM((CHR, D), jnp.float32),
            pltpu.VMEM((CHR, D), jnp.float32),
            pltpu.VMEM((DP,), jnp.float32),
            pltpu.VMEM((DP,), jnp.float32),
            pltpu.VMEM((DP,), jnp.float32),
            pltpu.VMEM((2 * B,), jnp.int32),
            pltpu.VMEM_SHARED((NS * DP,), jnp.float32),
            pltpu.SemaphoreType.DMA,
            pltpu.SemaphoreType.DMA,
        ],
    )(seq2d, len32)


def kernel(sequences, lengths):
    seq2d = sequences.reshape(B * L, D)
    len32 = lengths.astype(jnp.int32)
    out = _mean_sc(seq2d, len32)
    return jnp.concatenate([out[:, : (NV - 1) * LANES], out[:, 292:]], axis=1)
